# baseline (speedup 1.0000x reference)
import functools

import jax
import jax.numpy as jnp
from jax import lax
from jax.experimental import pallas as pl
from jax.experimental.pallas import tpu as pltpu
from jax.experimental.pallas import tpu_sc as plsc

B = 16384
NF = 32
GPR = 128 // NF
NC, NS = 2, 16
NW = NC * NS
BPW = B // NW
CHUNK = 64
NCH = BPW // CHUNK
TB = 2048


def _sc_gather(uidx, iidx, udiv, idiv, du_t, di_t, wu_t, wi_t):
    mesh = plsc.VectorSubcoreMesh(core_axis_name="c", subcore_axis_name="s")
    f32 = jnp.float32
    out = jax.ShapeDtypeStruct((B, 128), f32)

    @functools.partial(
        pl.kernel,
        mesh=mesh,
        out_type=(out, out, out, out),
        scratch_types=(
            [pltpu.VMEM((BPW // 128, 128), jnp.int32) for _ in range(4)]
            + [pltpu.VMEM((CHUNK, 128), f32) for _ in range(8)]
            + [pltpu.SemaphoreType.DMA, pltpu.SemaphoreType.DMA,
               pltpu.SemaphoreType.DMA]
        ),
    )
    def k(uidx_h, iidx_h, udiv_h, idiv_h, du_h, di_h, wu_h, wi_h,
          duo_h, dio_h, wuo_h, wio_h,
          uix_v, iix_v, udv_v, idv_v,
          du_a, di_a, wu_a, wi_a, du_b, di_b, wu_b, wi_b,
          gsem, osem_a, osem_b):
        wid = lax.axis_index("s") * NC + lax.axis_index("c")
        base = wid * BPW
        du_r = du_h
        di_r = di_h
        nr = BPW // 128
        pltpu.sync_copy(uidx_h.at[pl.ds(wid * nr, nr)], uix_v)
        pltpu.sync_copy(iidx_h.at[pl.ds(wid * nr, nr)], iix_v)
        pltpu.sync_copy(udiv_h.at[pl.ds(wid * nr, nr)], udv_v)
        pltpu.sync_copy(idiv_h.at[pl.ds(wid * nr, nr)], idv_v)
        sets = (
            ((du_a, di_a, wu_a, wi_a), osem_a),
            ((du_b, di_b, wu_b, wi_b), osem_b),
        )
        pending = [[], []]
        for c in range(NCH):
            s = c % 2
            (bufs, osem) = sets[s]
            dub, dib, wub, wib = bufs
            for h in pending[s]:
                h.wait()
            pending[s] = []
            r, half = c // 2, (c % 2) * CHUNK
            g = [
                pltpu.async_copy(du_r.at[uix_v.at[r, pl.ds(half, CHUNK)]], dub, gsem),
                pltpu.async_copy(di_r.at[iix_v.at[r, pl.ds(half, CHUNK)]], dib, gsem),
                pltpu.async_copy(wu_h.at[udv_v.at[r, pl.ds(half, CHUNK)]], wub, gsem),
                pltpu.async_copy(wi_h.at[idv_v.at[r, pl.ds(half, CHUNK)]], wib, gsem),
            ]
            for h in g:
                h.wait()
            sl = pl.ds(base + c * CHUNK, CHUNK)
            pending[s] = [
                pltpu.async_copy(dub, duo_h.at[sl], osem),
                pltpu.async_copy(dib, dio_h.at[sl], osem),
                pltpu.async_copy(wub, wuo_h.at[sl], osem),
                pltpu.async_copy(wib, wio_h.at[sl], osem),
            ]
        for s in range(2):
            for h in pending[s]:
                h.wait()

    return k(uidx, iidx, udiv, idiv, du_t, di_t, wu_t, wi_t)


def _mlp_body(du, di, wu, wi, m4u, m4i, mlu, mli,
              w1a, w1b, b1, w2, b2, wod, wmisc, o):
    f32 = jnp.float32
    du128 = du[...]
    di128 = di[...]
    g4u = m4u[...]
    g4i = m4i[...]
    xu = jnp.zeros((TB, NF), f32)
    xi = jnp.zeros((TB, NF), f32)
    for g in range(GPR):
        sl = slice(g * NF, (g + 1) * NF)
        xu = xu + jnp.where(g4u == g, du128[:, sl], 0.0)
        xi = xi + jnp.where(g4i == g, di128[:, sl], 0.0)
    h = jnp.dot(xu, w1a[...], preferred_element_type=f32)
    h = h + jnp.dot(xi, w1b[...], preferred_element_type=f32)
    h = jnp.maximum(h + b1[...], 0.0)
    h = jnp.maximum(jnp.dot(h, w2[...], preferred_element_type=f32) + b2[...],
                    0.0)
    out = jnp.dot(h, wod[...], preferred_element_type=f32)
    lane = lax.broadcasted_iota(jnp.int32, (TB, 128), 1)
    wuv = jnp.sum(jnp.where(lane == mlu[...], wu[...], 0.0),
                  axis=1, keepdims=True)
    wiv = jnp.sum(jnp.where(lane == mli[...], wi[...], 0.0),
                  axis=1, keepdims=True)
    o[...] = out + wuv * wmisc[0, 0] + wiv * wmisc[0, 1] + wmisc[0, 2]


def _mlp(du, di, wu, wi, m4u, m4i, mlu, mli, w1a, w1b, b1, w2, b2, wod, wmisc):
    row = lambda i: (i, 0)
    fixed = lambda i: (0, 0)
    wide = pl.BlockSpec((TB, 128), row)
    narrow = pl.BlockSpec((TB, 1), row)
    return pl.pallas_call(
        _mlp_body,
        grid=(B // TB,),
        in_specs=[
            wide, wide, wide, wide,
            narrow, narrow, narrow, narrow,
            pl.BlockSpec((NF, 64), fixed),
            pl.BlockSpec((NF, 64), fixed),
            pl.BlockSpec((1, 64), fixed),
            pl.BlockSpec((64, 32), fixed),
            pl.BlockSpec((1, 32), fixed),
            pl.BlockSpec((32, 1), fixed),
            pl.BlockSpec((1, 3), fixed),
        ],
        out_specs=narrow,
        out_shape=jax.ShapeDtypeStruct((B, 1), jnp.float32),
    )(du, di, wu, wi, m4u, m4i, mlu, mli, w1a, w1b, b1, w2, b2, wod, wmisc)


def kernel(user_idx, item_idx, wide_user_w, wide_item_w,
           deep_user_w, deep_item_w, W1, b1, W2, b2, Wo, bo):
    f32 = jnp.float32
    ui = user_idx.astype(jnp.int32)
    ii = item_idx.astype(jnp.int32)
    uidx = (ui // GPR).reshape(B // 128, 128)
    iidx = (ii // GPR).reshape(B // 128, 128)
    udiv = (ui // 128).reshape(B // 128, 128)
    idiv = (ii // 128).reshape(B // 128, 128)
    m4u = (ui % GPR).reshape(B, 1)
    m4i = (ii % GPR).reshape(B, 1)
    mlu = (ui % 128).reshape(B, 1)
    mli = (ii % 128).reshape(B, 1)

    def pad128(w):
        n = w.shape[0]
        rows = -(-n // 128)
        flat = w.reshape(n)
        return jnp.concatenate(
            [flat, jnp.zeros((rows * 128 - n,), f32)]).reshape(rows, 128)

    wu_t = pad128(wide_user_w)
    wi_t = pad128(wide_item_w)

    du, di, wu, wi = _sc_gather(
        uidx, iidx, udiv, idiv,
        deep_user_w.reshape(-1, 128), deep_item_w.reshape(-1, 128),
        wu_t, wi_t)

    wod = Wo[2:]
    wmisc = jnp.concatenate(
        [Wo[0, 0].reshape(1), Wo[1, 0].reshape(1), bo]).reshape(1, 3)
    out = _mlp(du, di, wu, wi, m4u, m4i, mlu, mli,
               W1[:NF], W1[NF:], b1.reshape(1, 64), W2, b2.reshape(1, 32),
               wod, wmisc)
    return out[:, 0]


# speedup vs baseline: 1.4345x; 1.4345x over previous
import dataclasses
import functools

import jax
import jax.numpy as jnp
from jax import lax
from jax.experimental import pallas as pl
from jax.experimental.pallas import tpu as pltpu
from jax.experimental.pallas import tpu_sc as plsc

B = 16384
NF = 32
NC, NS = 2, 16
NW = NC * NS
BPW = B // NW
CHUNK = 64
NCH = BPW // CHUNK
TB = 2048
IDXR = BPW // 128


def _sc_gather(uidx, iidx, udiv, idiv, umod, imod, du_t, di_t, wu_t, wi_t):
    mesh = plsc.VectorSubcoreMesh(core_axis_name="c", subcore_axis_name="s")
    f32 = jnp.float32
    i8 = jnp.int8
    i32 = jnp.int32
    cp = pltpu.CompilerParams()
    if "needs_layout_passes" in pltpu.CompilerParams.__dataclass_fields__:
        cp = dataclasses.replace(cp, needs_layout_passes=False)

    @functools.partial(
        pl.kernel,
        mesh=mesh,
        compiler_params=cp,
        out_type=(
            jax.ShapeDtypeStruct((B, NF), f32),
            jax.ShapeDtypeStruct((B, NF), f32),
            jax.ShapeDtypeStruct((B // 128, 128), f32),
            jax.ShapeDtypeStruct((B // 128, 128), f32),
        ),
        scratch_types=(
            [pltpu.VMEM((IDXR, 128), i32) for _ in range(6)]
            + [pltpu.VMEM((CHUNK, NF), f32) for _ in range(4)]
            + [pltpu.VMEM((CHUNK, 128), f32) for _ in range(4)]
            + [pltpu.VMEM((IDXR, 128), f32) for _ in range(2)]
            + [pltpu.SemaphoreType.DMA, pltpu.SemaphoreType.DMA,
               pltpu.SemaphoreType.DMA, pltpu.SemaphoreType.DMA]
        ),
    )
    def k(uidx_h, iidx_h, udiv_h, idiv_h, umod_h, imod_h,
          du_h, di_h, wu_h, wi_h,
          duo_h, dio_h, wuo_h, wio_h,
          udv_v, idv_v, umd_v, imd_v, uraw_v, iraw_v,
          du_a, di_a, du_b, di_b,
          wu_a, wi_a, wu_b, wi_b,
          wuo_v, wio_v,
          gsem, dsem, osem_a, osem_b):
        wid = lax.axis_index("s") * NC + lax.axis_index("c")
        base = wid * BPW
        for src, dst in ((uidx_h, uraw_v), (iidx_h, iraw_v),
                         (udiv_h, udv_v), (idiv_h, idv_v),
                         (umod_h, umd_v), (imod_h, imd_v)):
            pltpu.sync_copy(src.at[pl.ds(wid * IDXR, IDXR)], dst)
        sets = (
            ((du_a, di_a, wu_a, wi_a), osem_a),
            ((du_b, di_b, wu_b, wi_b), osem_b),
        )
        pending = [[], []]
        jvec = lax.iota(i32, 16)
        for c in range(NCH):
            s = c % 2
            bufs, osem = sets[s]
            dub, dib, wub, wib = bufs
            for h in pending[s]:
                h.wait()
            pending[s] = []
            r, half = c // 2, (c % 2) * CHUNK
            g = [
                pltpu.async_copy(wu_h.at[udv_v.at[r, pl.ds(half, CHUNK)]],
                                 wub, gsem),
                pltpu.async_copy(wi_h.at[idv_v.at[r, pl.ds(half, CHUNK)]],
                                 wib, gsem),
            ]

            for gq in range(CHUNK // 16):
                vu = uraw_v.at[r, pl.ds(half + 16 * gq, 16)][...]
                vi = iraw_v.at[r, pl.ds(half + 16 * gq, 16)][...]
                for l in range(16):
                    j = 16 * gq + l
                    pltpu.async_copy(du_h.at[vu[l]], dub.at[j], dsem)
                    pltpu.async_copy(di_h.at[vi[l]], dib.at[j], dsem)

            pltpu.make_async_copy(du_h.at[pl.ds(0, CHUNK)], dub, dsem).wait()
            pltpu.make_async_copy(di_h.at[pl.ds(0, CHUNK)], dib, dsem).wait()
            for h in g:
                h.wait()
            orow, ocol = c // 2, (c % 2) * CHUNK
            for gi in range(CHUNK // 16):
                jj = jvec + 16 * gi
                ucols = umd_v.at[r, pl.ds(half + 16 * gi, 16)][...]
                icols = imd_v.at[r, pl.ds(half + 16 * gi, 16)][...]
                usel = plsc.load_gather(wub, [jj, ucols])
                isel = plsc.load_gather(wib, [jj, icols])
                wuo_v.at[orow, pl.ds(ocol + 16 * gi, 16)][...] = usel
                wio_v.at[orow, pl.ds(ocol + 16 * gi, 16)][...] = isel
            sl = pl.ds(base + c * CHUNK, CHUNK)
            pending[s] = [
                pltpu.async_copy(dub, duo_h.at[sl], osem),
                pltpu.async_copy(dib, dio_h.at[sl], osem),
            ]
        for s in range(2):
            for h in pending[s]:
                h.wait()
        pltpu.sync_copy(wuo_v, wuo_h.at[pl.ds(wid * IDXR, IDXR)])
        pltpu.sync_copy(wio_v, wio_h.at[pl.ds(wid * IDXR, IDXR)])

    return k(uidx, iidx, udiv, idiv, umod, imod, du_t, di_t, wu_t, wi_t)


def _mlp_body(du, di, wu, wi, w1a, w1b, b1, w2, b2, wod, wmisc, o):
    f32 = jnp.float32
    h = jnp.dot(du[...], w1a[...], preferred_element_type=f32)
    h = h + jnp.dot(di[...], w1b[...], preferred_element_type=f32)
    h = jnp.maximum(h + b1[...], 0.0)
    h = jnp.maximum(jnp.dot(h, w2[...], preferred_element_type=f32) + b2[...],
                    0.0)
    out = jnp.dot(h, wod[...], preferred_element_type=f32)
    o[...] = out + wu[...] * wmisc[0, 0] + wi[...] * wmisc[0, 1] + wmisc[0, 2]


def _mlp(du, di, wu, wi, w1a, w1b, b1, w2, b2, wod, wmisc):
    row = lambda i: (i, 0)
    fixed = lambda i: (0, 0)
    narrow = pl.BlockSpec((TB, 1), row)
    emb = pl.BlockSpec((TB, NF), row)
    return pl.pallas_call(
        _mlp_body,
        grid=(B // TB,),
        in_specs=[
            emb, emb, narrow, narrow,
            pl.BlockSpec((NF, 64), fixed),
            pl.BlockSpec((NF, 64), fixed),
            pl.BlockSpec((1, 64), fixed),
            pl.BlockSpec((64, 32), fixed),
            pl.BlockSpec((1, 32), fixed),
            pl.BlockSpec((32, 1), fixed),
            pl.BlockSpec((1, 3), fixed),
        ],
        out_specs=narrow,
        out_shape=jax.ShapeDtypeStruct((B, 1), jnp.float32),
    )(du, di, wu, wi, w1a, w1b, b1, w2, b2, wod, wmisc)


def kernel(user_idx, item_idx, wide_user_w, wide_item_w,
           deep_user_w, deep_item_w, W1, b1, W2, b2, Wo, bo):
    ui = user_idx.astype(jnp.int32)
    ii = item_idx.astype(jnp.int32)
    uidx = ui.reshape(B // 128, 128)
    iidx = ii.reshape(B // 128, 128)

    def pad128(w):
        n = w.shape[0]
        rows = -(-n // 128)
        flat = w.reshape(n)
        return jnp.concatenate(
            [flat, jnp.zeros((rows * 128 - n,), jnp.float32)]).reshape(rows, 128)

    du, di, wu, wi = _sc_gather(
        uidx, iidx, uidx // 128, iidx // 128, uidx % 128, iidx % 128,
        deep_user_w, deep_item_w,
        pad128(wide_user_w), pad128(wide_item_w))

    wod = Wo[2:]
    wmisc = jnp.concatenate(
        [Wo[0, 0].reshape(1), Wo[1, 0].reshape(1), bo]).reshape(1, 3)
    out = _mlp(du, di, wu.reshape(B, 1), wi.reshape(B, 1),
               W1[:NF], W1[NF:], b1.reshape(1, 64), W2, b2.reshape(1, 32),
               wod, wmisc)
    return out[:, 0]
